# Initial kernel scaffold; baseline (speedup 1.0000x reference)
import jax
import jax.numpy as jnp
from jax.experimental import pallas as pl


def kernel(piece_ids, orientations, corner_slot_w, corner_piece_w, corner_orient_w, edge_slot_w, edge_piece_w, edge_orient_w, proj_w, proj_b):
    raise NotImplementedError("write your pallas kernel here")


# speedup vs baseline: 1.2348x; 1.2348x over previous
import jax
import jax.numpy as jnp
from jax.experimental import pallas as pl


def _fused_kernel(pid_ref, orient_ref, slot_all_ref, piece_all_ref,
                  orient_all_ref, proj_w_ref, proj_b_ref, out_ref):
    pid = pid_ref[...]
    oid = orient_ref[...]

    row = jax.lax.broadcasted_iota(jnp.int32, (20, 1), 0)
    oid_adj = oid + jnp.where(row >= 8, 3, 0)

    k20 = jax.lax.broadcasted_iota(jnp.int32, (20, 20), 1)
    onehot_p = (pid == k20).astype(jnp.float32)
    emb_piece = jnp.dot(onehot_p, piece_all_ref[...],
                        preferred_element_type=jnp.float32)

    k5 = jax.lax.broadcasted_iota(jnp.int32, (20, 5), 1)
    onehot_o = (oid_adj == k5).astype(jnp.float32)
    emb_orient = jnp.dot(onehot_o, orient_all_ref[...],
                         preferred_element_type=jnp.float32)

    embedded = jnp.concatenate(
        [slot_all_ref[...], emb_piece, emb_orient], axis=1)

    out_ref[...] = (jnp.dot(embedded, proj_w_ref[...],
                            preferred_element_type=jnp.float32)
                    + proj_b_ref[...])


def kernel(piece_ids, orientations, corner_slot_w, corner_piece_w,
           corner_orient_w, edge_slot_w, edge_piece_w, edge_orient_w,
           proj_w, proj_b):
    pid = piece_ids.reshape(20, 1)
    oid = orientations.reshape(20, 1)
    slot_all = jnp.concatenate([corner_slot_w, edge_slot_w], axis=0)
    piece_all = jnp.concatenate([corner_piece_w, edge_piece_w], axis=0)
    orient_all = jnp.concatenate([corner_orient_w, edge_orient_w], axis=0)

    out = pl.pallas_call(
        _fused_kernel,
        out_shape=jax.ShapeDtypeStruct((20, 256), jnp.float32),
    )(pid, oid, slot_all, piece_all, orient_all, proj_w,
      proj_b.reshape(1, 256))
    return out.reshape(1, 20, 256)
